# baseline (speedup 1.0000x reference)
import functools

import jax
import jax.numpy as jnp
from jax import lax
from jax.experimental import pallas as pl
from jax.experimental.pallas import tpu as pltpu
from jax.experimental.pallas import tpu_sc as plsc

L = 16


@functools.cache
def _build(B, V, E):
    info = plsc.get_sparse_core_info()
    NC, NS = info.num_cores, info.num_subcores
    NW = NC * NS
    b_w = B // NW
    G = b_w // L
    mesh = plsc.VectorSubcoreMesh(core_axis_name="c", subcore_axis_name="s")

    @functools.partial(
        pl.kernel,
        mesh=mesh,
        out_type=jax.ShapeDtypeStruct((NC, L), jnp.float32),
        compiler_params=pltpu.CompilerParams(
            needs_layout_passes=False, use_tc_tiling_on_sc=False),
        scratch_types=[
            pltpu.VMEM((b_w,), jnp.int32),
            pltpu.VMEM((b_w,), jnp.int32),
            pltpu.VMEM((b_w,), jnp.float32),
            pltpu.VMEM((b_w, E), jnp.float32),
            pltpu.VMEM((b_w, E), jnp.float32),
            pltpu.VMEM((b_w,), jnp.float32),
            pltpu.VMEM((b_w,), jnp.float32),
            pltpu.VMEM((L,), jnp.float32),
            pltpu.VMEM((NS, L), jnp.float32),
            pltpu.VMEM_SHARED((NS, L), jnp.float32),
            pltpu.SemaphoreType.DMA,
            pltpu.SemaphoreType.DMA,
            pltpu.SemaphoreType.DMA,
            pltpu.SemaphoreType.DMA,
        ],
    )
    def glove_kernel(cw_hbm, tw_hbm, w_hbm, ev_hbm, eu_hbm, vb_hbm, ub_hbm,
                     out_hbm, idx_c, idx_t, w_v, rows_v, rows_u, bias_c,
                     bias_t, res_v, part_v, shared, sem0, sem1, sem2, sem3):
        cid = lax.axis_index("c")
        sid = lax.axis_index("s")
        wid = sid * NC + cid
        base = wid * b_w

        pltpu.sync_copy(cw_hbm.at[pl.ds(base, b_w)], idx_c)
        pltpu.sync_copy(tw_hbm.at[pl.ds(base, b_w)], idx_t)
        pltpu.sync_copy(w_hbm.at[pl.ds(base, b_w)], w_v)
        cp0 = pltpu.async_copy(ev_hbm.at[idx_c], rows_v, sem0)
        cp1 = pltpu.async_copy(eu_hbm.at[idx_t], rows_u, sem1)
        cp2 = pltpu.async_copy(vb_hbm.at[idx_c], bias_c, sem2)
        cp3 = pltpu.async_copy(ub_hbm.at[idx_t], bias_t, sem3)
        cp0.wait()
        cp1.wait()
        cp2.wait()
        cp3.wait()

        lane = lax.iota(jnp.int32, L)

        def group(g, acc):
            rid = g * L + lane
            ip = jnp.zeros((L,), jnp.float32)
            for e in range(E):
                col = jnp.full((L,), e, jnp.int32)
                cv = plsc.load_gather(rows_v, [rid, col])
                tv = plsc.load_gather(rows_u, [rid, col])
                ip = ip + cv * tv
            cb = bias_c[pl.ds(g * L, L)]
            tb = bias_t[pl.ds(g * L, L)]
            w = w_v[pl.ds(g * L, L)]
            t = ip + cb + tb
            return acc + w * t * t

        acc = lax.fori_loop(0, G, group, jnp.zeros((L,), jnp.float32))

        res_v[...] = acc
        pltpu.sync_copy(res_v, shared.at[sid])
        plsc.subcore_barrier()

        @pl.when(sid == 0)
        def _():
            pltpu.sync_copy(shared, part_v)
            tot = jnp.zeros((L,), jnp.float32)
            for s in range(NS):
                tot = tot + part_v[s]
            total = jnp.sum(tot)
            res_v[...] = jnp.full((L,), total, jnp.float32)
            pltpu.sync_copy(res_v, out_hbm.at[cid])

    return glove_kernel


def kernel(center_words, target_words, co_occurrences, weighting,
           emb_v, emb_u, v_bias, u_bias):
    B = center_words.shape[0]
    V, E = emb_v.shape
    cw = center_words.reshape(B)
    tw = target_words.reshape(B)
    w = weighting.reshape(B)
    vb = v_bias.reshape(V)
    ub = u_bias.reshape(V)
    out = _build(B, V, E)(cw, tw, w, emb_v, emb_u, vb, ub)
    return out[0, 0] + out[1, 0]
